# baseline (device time: 21852 ns/iter reference)
import jax
import jax.numpy as jnp
from jax import lax
from jax.experimental import pallas as pl
from jax.experimental.pallas import tpu as pltpu

N_SLICES = 8
N_DEV = 16
NB = 4


def kernel(Q, K, V):
    b, kv, h, d = K.shape
    hh = h // N_SLICES
    hd2 = hh * d
    scale = d ** -0.5
    n_step = b // NB
    rows = NB * hh

    deltas = [
        (dx, dy, dz)
        for dx in (0, 1) for dy in (0, 1) for dz in (0, 1, 2, 3)
        if (dx, dy, dz) != (0, 0, 0)
    ]

    def body(s_ref, q_ref, k_ref, v_ref, o_ref, pbuf, gs_sem, gr_sem):
        i = pl.program_id(0)
        my_x = lax.axis_index("x")
        my_y = lax.axis_index("y")
        my_z = lax.axis_index("z")
        my_r = s_ref[0]
        sid = my_x * N_SLICES + my_r
        off = my_r * hh
        peers = [
            (my_x ^ dx, my_y ^ dy, (my_z + dz) % 4) for dx, dy, dz in deltas
        ]

        @pl.when(i == 0)
        def _():
            barrier = pltpu.get_barrier_semaphore()
            for prt in peers:
                pl.semaphore_signal(
                    barrier, inc=1, device_id=prt,
                    device_id_type=pl.DeviceIdType.MESH,
                )
            pl.semaphore_wait(barrier, len(peers))

        eyef = (
            lax.broadcasted_iota(jnp.int32, (rows, rows), 0)
            == lax.broadcasted_iota(jnp.int32, (rows, rows), 1)
        ).astype(jnp.float32)
        q2 = q_ref[:, 0, pl.ds(off, hh), :].reshape(rows, d)
        qbd = (q2[:, None, :] * eyef[:, :, None]).reshape(rows, rows * d)
        k2 = k_ref[...].reshape(rows * d, kv).astype(jnp.bfloat16)
        s = lax.dot_general(
            qbd.astype(jnp.bfloat16), k2,
            (((1,), (0,)), ((), ())),
            preferred_element_type=jnp.float32,
        ) * scale
        m = jnp.max(s, axis=-1, keepdims=True)
        p = jnp.exp(s - m)
        l = jnp.sum(p, axis=-1, keepdims=True)
        v2 = v_ref[...].reshape(rows * d, kv).astype(jnp.bfloat16)
        r = lax.dot_general(
            p.astype(jnp.bfloat16), v2,
            (((1,), (1,)), ((), ())),
            preferred_element_type=jnp.float32,
        )
        o = jnp.sum(r.reshape(rows, rows, d) * eyef[:, :, None], axis=1)

        pbuf[sid, pl.ds(i * rows, rows), :] = o
        for w in range(n_step):
            @pl.when(i == w)
            def _(w=w):
                pbuf[sid, pl.ds(2 * b, 1), pl.ds(w * rows, rows)] = (
                    m.reshape(1, rows)
                )
                pbuf[sid, pl.ds(2 * b, 1), pl.ds(2 * b + w * rows, rows)] = (
                    l.reshape(1, rows)
                )

        @pl.when(i == n_step - 1)
        def _():
            for j, prt in enumerate(peers):
                pltpu.make_async_remote_copy(
                    src_ref=pbuf.at[pl.ds(sid, 1)],
                    dst_ref=pbuf.at[pl.ds(sid, 1)],
                    send_sem=gs_sem.at[j], recv_sem=gr_sem.at[j],
                    device_id=prt, device_id_type=pl.DeviceIdType.MESH,
                ).start()
            for j, prt in enumerate(peers):
                pltpu.make_async_remote_copy(
                    src_ref=pbuf.at[pl.ds(0, 1)],
                    dst_ref=pbuf.at[pl.ds(0, 1)],
                    send_sem=gs_sem.at[j], recv_sem=gr_sem.at[j],
                    device_id=prt, device_id_type=pl.DeviceIdType.MESH,
                ).wait_recv()

            pall = pbuf[...].reshape(2, N_SLICES, 2 * b + 2, d)
            o_half = pall[:, :, :2 * b, :].reshape(2, N_SLICES, b, hh, d)
            stats = pall[:, :, 2 * b, :]
            m_half = stats[:, :, :2 * b].reshape(2, N_SLICES, b, hh)
            l_half = stats[:, :, 2 * b:4 * b].reshape(2, N_SLICES, b, hh)
            m_new = jnp.maximum(m_half[0], m_half[1])
            a0 = jnp.exp(m_half[0] - m_new)[..., None]
            a1 = jnp.exp(m_half[1] - m_new)[..., None]
            l_new = l_half[0][..., None] * a0 + l_half[1][..., None] * a1
            o_fin = (o_half[0] * a0 + o_half[1] * a1) / l_new

            for j, prt in enumerate(peers):
                pltpu.make_async_remote_copy(
                    src_ref=pbuf.at[pl.ds(sid, 1)],
                    dst_ref=pbuf.at[pl.ds(sid, 1)],
                    send_sem=gs_sem.at[j], recv_sem=gr_sem.at[j],
                    device_id=prt, device_id_type=pl.DeviceIdType.MESH,
                ).wait_send()

            o_ref[...] = o_fin.transpose(1, 0, 2, 3).reshape(b, h, d)[:, None]

    grid_spec = pltpu.PrefetchScalarGridSpec(
        num_scalar_prefetch=1,
        grid=(n_step,),
        in_specs=[
            pl.BlockSpec((NB, 1, h, d), lambda i, s: (i, 0, 0, 0)),
            pl.BlockSpec((NB, hh, d, kv), lambda i, s: (i, s[0], 0, 0)),
            pl.BlockSpec((NB, hh, d, kv), lambda i, s: (i, s[0], 0, 0)),
        ],
        out_specs=pl.BlockSpec((b, 1, h, d), lambda i, s: (0, 0, 0, 0)),
        scratch_shapes=[
            pltpu.VMEM((N_DEV, 2 * b + 2, d), jnp.float32),
            pltpu.SemaphoreType.DMA((len(deltas),)),
            pltpu.SemaphoreType.DMA((len(deltas),)),
        ],
    )

    ridx = (lax.axis_index("y") * 4 + lax.axis_index("z")).astype(jnp.int32)
    return pl.pallas_call(
        body,
        grid_spec=grid_spec,
        out_shape=jax.ShapeDtypeStruct((b, 1, h, d), jnp.float32),
        compiler_params=pltpu.CompilerParams(
            collective_id=0,
            dimension_semantics=("arbitrary",),
            vmem_limit_bytes=64 * 1024 * 1024,
        ),
    )(
        jnp.reshape(ridx, (1,)),
        Q,
        K.transpose(0, 2, 3, 1),
        V.transpose(0, 2, 3, 1),
    )


# device time: 21103 ns/iter; 1.0355x vs baseline; 1.0355x over previous
import jax
import jax.numpy as jnp
from jax import lax
from jax.experimental import pallas as pl
from jax.experimental.pallas import tpu as pltpu

N_SLICES = 8
NB = 4


def kernel(Q, K, V):
    b, kv, h, d = K.shape
    hh = h // N_SLICES
    scale = d ** -0.5
    n_step = b // NB
    rows = NB * hh

    deltas = [
        (dy, dz) for dy in (0, 1) for dz in (0, 1, 2, 3) if (dy, dz) != (0, 0)
    ]

    def body(s_ref, q_ref, k_ref, v_ref, o_ref,
             axc, g, xs_sem, xr_sem, gs_sem, gr_sem):
        i = pl.program_id(0)
        my_x = lax.axis_index("x")
        my_y = lax.axis_index("y")
        my_z = lax.axis_index("z")
        x_peer = (1 - my_x, my_y, my_z)
        g_peers = [(my_x, my_y ^ dy, (my_z + dz) % 4) for dy, dz in deltas]
        off = s_ref[0] * hh

        @pl.when(i == 0)
        def _():
            barrier = pltpu.get_barrier_semaphore()
            for prt in [x_peer] + g_peers:
                pl.semaphore_signal(
                    barrier, inc=1, device_id=prt,
                    device_id_type=pl.DeviceIdType.MESH,
                )
            pl.semaphore_wait(barrier, 1 + len(g_peers))

        eyef = (
            lax.broadcasted_iota(jnp.int32, (rows, rows), 0)
            == lax.broadcasted_iota(jnp.int32, (rows, rows), 1)
        ).astype(jnp.float32)
        q2 = q_ref[:, 0, pl.ds(off, hh), :].reshape(rows, d)
        qbd = (q2[:, None, :] * eyef[:, :, None]).reshape(rows, rows * d)
        k2 = k_ref[...].reshape(rows * d, kv).astype(jnp.bfloat16)
        s = lax.dot_general(
            qbd.astype(jnp.bfloat16), k2,
            (((1,), (0,)), ((), ())),
            preferred_element_type=jnp.float32,
        ) * scale
        m = jnp.max(s, axis=-1, keepdims=True)
        p = jnp.exp(s - m)
        l = jnp.sum(p, axis=-1, keepdims=True)
        v2 = v_ref[...].reshape(rows * d, kv).astype(jnp.bfloat16)
        r = lax.dot_general(
            p.astype(jnp.bfloat16), v2,
            (((1,), (1,)), ((), ())),
            preferred_element_type=jnp.float32,
        )
        o = jnp.sum(r.reshape(rows, rows, d) * eyef[:, :, None], axis=1)

        axc[0, pl.ds(i * NB, NB)] = o.reshape(NB, hh, d)
        for w in range(n_step):
            @pl.when(i == w)
            def _(w=w):
                axc[0, pl.ds(b, 1), :, pl.ds(2 * NB * w, NB)] = (
                    m.reshape(NB, hh).T[None]
                )
                axc[0, pl.ds(b, 1), :, pl.ds(2 * NB * w + NB, NB)] = (
                    l.reshape(NB, hh).T[None]
                )
                pltpu.make_async_remote_copy(
                    src_ref=axc.at[0, pl.ds(w * NB, NB)],
                    dst_ref=axc.at[1, pl.ds(w * NB, NB)],
                    send_sem=xs_sem.at[w], recv_sem=xr_sem.at[w],
                    device_id=x_peer, device_id_type=pl.DeviceIdType.MESH,
                ).start()
                if w == n_step - 1:
                    pltpu.make_async_remote_copy(
                        src_ref=axc.at[0, pl.ds(b, 1)],
                        dst_ref=axc.at[1, pl.ds(b, 1)],
                        send_sem=xs_sem.at[n_step], recv_sem=xr_sem.at[n_step],
                        device_id=x_peer, device_id_type=pl.DeviceIdType.MESH,
                    ).start()

        @pl.when(i == n_step - 1)
        def _():
            for w in range(n_step):
                pltpu.make_async_remote_copy(
                    src_ref=axc.at[0, pl.ds(w * NB, NB)],
                    dst_ref=axc.at[1, pl.ds(w * NB, NB)],
                    send_sem=xs_sem.at[w], recv_sem=xr_sem.at[w],
                    device_id=x_peer, device_id_type=pl.DeviceIdType.MESH,
                ).wait_recv()
            pltpu.make_async_remote_copy(
                src_ref=axc.at[0, pl.ds(b, 1)],
                dst_ref=axc.at[1, pl.ds(b, 1)],
                send_sem=xs_sem.at[n_step], recv_sem=xr_sem.at[n_step],
                device_id=x_peer, device_id_type=pl.DeviceIdType.MESH,
            ).wait_recv()

            def unpack(slab):
                ms = jnp.concatenate(
                    [slab[:, 2 * NB * w: 2 * NB * w + NB] for w in range(n_step)],
                    axis=1,
                )
                ls = jnp.concatenate(
                    [slab[:, 2 * NB * w + NB: 2 * NB * (w + 1)] for w in range(n_step)],
                    axis=1,
                )
                return ms, ls

            m_loc, l_loc = unpack(axc[0, b])
            m_rem, l_rem = unpack(axc[1, b])
            m_new = jnp.maximum(m_loc, m_rem)
            a_loc = jnp.exp(m_loc - m_new).T[:, :, None]
            a_rem = jnp.exp(m_rem - m_new).T[:, :, None]
            l_new = (l_loc * jnp.exp(m_loc - m_new)
                     + l_rem * jnp.exp(m_rem - m_new)).T[:, :, None]
            o_fin = (
                axc[0, pl.ds(0, b)] * a_loc + axc[1, pl.ds(0, b)] * a_rem
            ) / l_new
            g[pl.ds(off, hh)] = o_fin.transpose(1, 0, 2)

            for j, prt in enumerate(g_peers):
                pltpu.make_async_remote_copy(
                    src_ref=g.at[pl.ds(off, hh)],
                    dst_ref=g.at[pl.ds(off, hh)],
                    send_sem=gs_sem.at[j], recv_sem=gr_sem.at[j],
                    device_id=prt, device_id_type=pl.DeviceIdType.MESH,
                ).start()
            for j, prt in enumerate(g_peers):
                pltpu.make_async_remote_copy(
                    src_ref=g.at[pl.ds(0, hh)],
                    dst_ref=g.at[pl.ds(0, hh)],
                    send_sem=gs_sem.at[j], recv_sem=gr_sem.at[j],
                    device_id=prt, device_id_type=pl.DeviceIdType.MESH,
                ).wait_recv()
            for j, prt in enumerate(g_peers):
                pltpu.make_async_remote_copy(
                    src_ref=g.at[pl.ds(off, hh)],
                    dst_ref=g.at[pl.ds(off, hh)],
                    send_sem=gs_sem.at[j], recv_sem=gr_sem.at[j],
                    device_id=prt, device_id_type=pl.DeviceIdType.MESH,
                ).wait_send()
            for w in range(n_step):
                pltpu.make_async_remote_copy(
                    src_ref=axc.at[0, pl.ds(w * NB, NB)],
                    dst_ref=axc.at[1, pl.ds(w * NB, NB)],
                    send_sem=xs_sem.at[w], recv_sem=xr_sem.at[w],
                    device_id=x_peer, device_id_type=pl.DeviceIdType.MESH,
                ).wait_send()
            pltpu.make_async_remote_copy(
                src_ref=axc.at[0, pl.ds(b, 1)],
                dst_ref=axc.at[1, pl.ds(b, 1)],
                send_sem=xs_sem.at[n_step], recv_sem=xr_sem.at[n_step],
                device_id=x_peer, device_id_type=pl.DeviceIdType.MESH,
            ).wait_send()

            o_ref[...] = g[...].transpose(1, 0, 2)[:, None]

    grid_spec = pltpu.PrefetchScalarGridSpec(
        num_scalar_prefetch=1,
        grid=(n_step,),
        in_specs=[
            pl.BlockSpec((NB, 1, h, d), lambda i, s: (i, 0, 0, 0)),
            pl.BlockSpec((NB, hh, d, kv), lambda i, s: (i, s[0], 0, 0)),
            pl.BlockSpec((NB, hh, d, kv), lambda i, s: (i, s[0], 0, 0)),
        ],
        out_specs=pl.BlockSpec((b, 1, h, d), lambda i, s: (0, 0, 0, 0)),
        scratch_shapes=[
            pltpu.VMEM((2, b + 1, hh, d), jnp.float32),
            pltpu.VMEM((h, b, d), jnp.float32),
            pltpu.SemaphoreType.DMA((n_step + 1,)),
            pltpu.SemaphoreType.DMA((n_step + 1,)),
            pltpu.SemaphoreType.DMA((len(deltas),)),
            pltpu.SemaphoreType.DMA((len(deltas),)),
        ],
    )

    ridx = (lax.axis_index("y") * 4 + lax.axis_index("z")).astype(jnp.int32)
    return pl.pallas_call(
        body,
        grid_spec=grid_spec,
        out_shape=jax.ShapeDtypeStruct((b, 1, h, d), jnp.float32),
        compiler_params=pltpu.CompilerParams(
            collective_id=0,
            dimension_semantics=("arbitrary",),
            vmem_limit_bytes=64 * 1024 * 1024,
        ),
    )(
        jnp.reshape(ridx, (1,)),
        Q,
        K.transpose(0, 2, 3, 1),
        V.transpose(0, 2, 3, 1),
    )
